# baseline (speedup 1.0000x reference)
import functools

import jax
import jax.numpy as jnp
from jax import lax
from jax.experimental import pallas as pl
from jax.experimental.pallas import tpu as pltpu
from jax.experimental.pallas import tpu_sc as plsc

NC, NS, LANES = 2, 16, 16
NW = NC * NS
CH = 64
WWIN = 30
MAXC = (WWIN * WWIN + CH - 1) // CH
PAD = MAXC * CH
PADW = PAD + 16
TCW = 29
TCC = 40
KTC = 48


def _sc_encode(tf, ef, idxr, meta, n_pairs, d):
    ppw = n_pairs // NW
    vpr = d // LANES
    grp = 16
    ngrp = vpr // grp
    mesh = plsc.VectorSubcoreMesh(core_axis_name="c", subcore_axis_name="s",
                                  num_cores=NC, num_subcores=NS)

    @functools.partial(
        pl.kernel,
        out_type=jax.ShapeDtypeStruct((n_pairs, 5, d), jnp.float32),
        mesh=mesh,
        scratch_types=[
            pltpu.VMEM((PADW,), jnp.int32),
            pltpu.VMEM((PADW,), jnp.int32),
            pltpu.VMEM((2 * CH, d), jnp.float32),
            pltpu.VMEM((2, d), jnp.float32),
            pltpu.VMEM((2, d), jnp.float32),
            pltpu.VMEM((2, d), jnp.float32),
            pltpu.VMEM((2, d), jnp.float32),
            pltpu.VMEM((1, d), jnp.float32),
            pltpu.VMEM((1, d), jnp.float32),
            pltpu.VMEM((16,), jnp.int32),
            pltpu.SemaphoreType.DMA,
            pltpu.SemaphoreType.DMA,
            pltpu.SemaphoreType.DMA,
            pltpu.SemaphoreType.DMA,
            pltpu.SemaphoreType.DMA,
            pltpu.SemaphoreType.DMA,
            pltpu.SemaphoreType.DMA,
        ],
    )
    def k(tf_h, ef_h, idxr_h, meta_h, enc_h,
          idx_a, idx_b, rows, cor_t_a, cor_t_b, cor_e_a, cor_e_b,
          racc_a, racc_b, meta_v,
          sem_idx_a, sem_idx_b, sem_ra, sem_rb, sem_cor,
          sem_out_a, sem_out_b):
        wid = lax.axis_index("s") * NC + lax.axis_index("c")
        pltpu.sync_copy(meta_h.at[pl.ds(wid * 16, 16)], meta_v)
        mv = meta_v[...]
        idx_bufs = (idx_a, idx_b)
        row_halves = (rows.at[pl.ds(0, CH)], rows.at[pl.ds(CH, CH)])
        cor_t = (cor_t_a, cor_t_b)
        cor_e = (cor_e_a, cor_e_b)
        racc = (racc_a, racc_b)
        sem_idx = (sem_idx_a, sem_idx_b)
        sem_rows = (sem_ra, sem_rb)
        sem_out = (sem_out_a, sem_out_b)
        neg = jnp.full((LANES,), -jnp.inf, jnp.float32)

        def out_writes(jb, p):
            return (
                pltpu.make_async_copy(cor_t[jb], enc_h.at[p, pl.ds(0, 2)],
                                      sem_out[jb]),
                pltpu.make_async_copy(racc[jb], enc_h.at[p, pl.ds(2, 1)],
                                      sem_out[jb]),
                pltpu.make_async_copy(cor_e[jb], enc_h.at[p, pl.ds(3, 2)],
                                      sem_out[jb]),
            )

        pltpu.async_copy(idxr_h.at[pl.ds(mv[0] * PADW, PADW)],
                         idx_bufs[0], sem_idx[0])
        pltpu.make_async_copy(idxr_h.at[pl.ds(mv[0] * PADW, PADW)],
                              idx_bufs[0], sem_idx[0]).wait()
        pltpu.async_copy(tf_h.at[idx_bufs[0].at[pl.ds(0, CH)]],
                         row_halves[0], sem_rows[0])
        gbase = jnp.int32(0)
        prev_p = [None, None]
        for j in range(ppw):
            jb = j % 2
            p = mv[j]
            cnt = mv[8 + j]
            idx_v = idx_bufs[jb]
            if j >= 2:
                for w in out_writes(jb, prev_p[jb]):
                    w.wait()
            pltpu.async_copy(tf_h.at[idx_v.at[pl.ds(PAD, 2)]],
                             cor_t[jb], sem_cor)
            pltpu.async_copy(ef_h.at[idx_v.at[pl.ds(PAD, 2)]],
                             cor_e[jb], sem_cor)
            if j + 1 < ppw:
                pltpu.async_copy(idxr_h.at[pl.ds(mv[j + 1] * PADW, PADW)],
                                 idx_bufs[(j + 1) % 2], sem_idx[(j + 1) % 2])
            for kk in range(vpr):
                racc[jb][0, pl.ds(kk * LANES, LANES)] = neg

            def accum(base):
                for g in range(ngrp):
                    off = g * grp * LANES
                    accs = tuple(racc[jb][0, pl.ds(off + kk * LANES, LANES)]
                                 for kk in range(grp))

                    def row_body(rr, a):
                        return tuple(
                            jnp.maximum(a[kk],
                                        rows[base + rr,
                                             pl.ds(off + kk * LANES, LANES)])
                            for kk in range(grp))

                    accs = lax.fori_loop(0, CH, row_body, accs)
                    for kk in range(grp):
                        racc[jb][0, pl.ds(off + kk * LANES, LANES)] = accs[kk]

            def chunk_body(ci, carry):
                parv = (gbase + ci) % 2
                for par in range(2):
                    @pl.when(parv == par)
                    def _():
                        cur, nxt = row_halves[par], row_halves[1 - par]
                        pltpu.make_async_copy(
                            tf_h.at[idx_v.at[pl.ds(ci * CH, CH)]],
                            cur, sem_rows[par]).wait()

                        @pl.when(ci + 1 < cnt)
                        def _():
                            pltpu.async_copy(
                                tf_h.at[idx_v.at[pl.ds((ci + 1) * CH, CH)]],
                                nxt, sem_rows[1 - par])

                        if j + 1 < ppw:
                            nidx = idx_bufs[(j + 1) % 2]

                            @pl.when(ci + 1 == cnt)
                            def _():
                                pltpu.make_async_copy(
                                    idxr_h.at[pl.ds(mv[j + 1] * PADW, PADW)],
                                    nidx, sem_idx[(j + 1) % 2]).wait()
                                pltpu.async_copy(
                                    tf_h.at[nidx.at[pl.ds(0, CH)]],
                                    nxt, sem_rows[1 - par])

                accum(parv * CH)
                return carry

            lax.fori_loop(0, cnt, chunk_body, 0)
            gbase = (gbase + cnt) % 2

            pltpu.make_async_copy(tf_h.at[idx_v.at[pl.ds(PAD, 2)]],
                                  cor_t[jb], sem_cor).wait()
            pltpu.make_async_copy(ef_h.at[idx_v.at[pl.ds(PAD, 2)]],
                                  cor_e[jb], sem_cor).wait()
            for w in out_writes(jb, p):
                w.start()
            prev_p[jb] = p

        for jb in range(2):
            for w in out_writes(jb, prev_p[jb]):
                w.wait()

    return k(tf, ef, idxr, meta)


def _tc_rect(Table, sp, n_pairs, d):
    K = sp.shape[0]

    def body(sp_ref, tf_ref, out_ref, buf, sems):
        i = pl.program_id(0)

        def desc(slot, idx):
            b = sp_ref[idx, 0]
            u0 = sp_ref[idx, 1]
            v0a = pl.multiple_of(sp_ref[idx, 2], 8)
            return pltpu.make_async_copy(
                tf_ref.at[b, pl.ds(u0, TCW), pl.ds(v0a, TCC), :],
                buf.at[slot], sems.at[slot])

        @pl.when(i == 0)
        def _():
            desc(0, 0).start()

        @pl.when(i + 1 < K)
        def _():
            desc((i + 1) % 2, i + 1).start()

        slot = i % 2
        desc(slot, i).wait()
        nr = sp_ref[i, 3]
        ncv = sp_ref[i, 4]
        coff = sp_ref[i, 6]
        blk = buf[slot]
        rmask = lax.broadcasted_iota(jnp.int32, (TCW, TCC, 1), 0) < nr
        ci = lax.broadcasted_iota(jnp.int32, (TCW, TCC, 1), 1)
        cmask = (ci >= coff) & (ci < coff + ncv)
        masked = jnp.where(rmask & cmask, blk, -jnp.inf)
        out_ref[...] = jnp.max(masked, axis=(0, 1))[None, None, :]

    grid_spec = pltpu.PrefetchScalarGridSpec(
        num_scalar_prefetch=1,
        grid=(K,),
        in_specs=[pl.BlockSpec(memory_space=pl.ANY)],
        out_specs=pl.BlockSpec((1, 1, d), lambda i, sp_ref: (sp_ref[i, 5], 0, 0)),
        scratch_shapes=[pltpu.VMEM((2, TCW, TCC, d), jnp.float32),
                        pltpu.SemaphoreType.DMA((2,))],
    )
    return pl.pallas_call(
        body, grid_spec=grid_spec,
        out_shape=jax.ShapeDtypeStruct((n_pairs, 1, d), jnp.float32))(sp, Table)


def _classify(enc3, rtc, tcf, w3, b2, lab2):
    n, nslab, _ = enc3.shape

    def body(enc_ref, rtc_ref, tcf_ref, w_ref, b_ref, lab_ref,
             loss_ref, logits_ref, pred_ref):
        logits = b_ref[...]
        for rr in range(nslab):
            x = enc_ref[:, rr, :]
            if rr == 2:
                x = jnp.where(tcf_ref[...] > 0, rtc_ref[:, 0, :], x)
            logits = logits + jnp.dot(x, w_ref[rr],
                                      preferred_element_type=jnp.float32)
        m = jnp.max(logits, axis=1, keepdims=True)
        lse = jnp.log(jnp.sum(jnp.exp(logits - m), axis=1, keepdims=True)) + m
        logp = logits - lse
        lab = lab_ref[...]
        valid = lab >= 0
        safe = jnp.where(valid, lab, 0)
        cols = lax.broadcasted_iota(jnp.int32, logits.shape, 1)
        nll = -jnp.sum(jnp.where(cols == safe, logp, 0.0), axis=1, keepdims=True)
        cnt = jnp.maximum(jnp.sum(valid.astype(jnp.int32)), 1).astype(jnp.float32)
        loss_ref[...] = (jnp.sum(jnp.where(valid, nll, 0.0)) / cnt).reshape(1, 1)
        logits_ref[...] = logits
        pred_ref[...] = jnp.argmax(logits, axis=1).astype(jnp.int32)[:, None]

    return pl.pallas_call(
        body,
        out_shape=(
            jax.ShapeDtypeStruct((1, 1), jnp.float32),
            jax.ShapeDtypeStruct((n, 4), jnp.float32),
            jax.ShapeDtypeStruct((n, 1), jnp.int32),
        ),
    )(enc3, rtc, tcf, w3, b2, lab2)


def kernel(Table, table_edge, pairs, labels, W_lin, b_lin):
    B, L, _, D = Table.shape
    P = pairs.shape[1]
    N = B * P
    ppw = N // NW

    tf = Table.reshape(B * L * L, D)
    ef = table_edge.reshape(B * L * L, D)

    pf = pairs.reshape(N, 4).astype(jnp.int32)
    s0, e0, s1, e1 = pf[:, 0], pf[:, 1], pf[:, 2], pf[:, 3]
    base = (jnp.arange(N, dtype=jnp.int32) // P) * (L * L)
    nr = jnp.clip(e0 - s0, 1, WWIN)
    ncc = jnp.clip(e1 - s1, 1, WWIN)
    n = nr * ncc
    kk = jnp.arange(PAD, dtype=jnp.int32)[None, :]
    magic_n = ((1 << 21) + n - 1) // n
    km = kk - n[:, None] * ((kk * magic_n[:, None]) >> 21)
    magic_c = ((1 << 18) + ncc - 1) // ncc
    r = (km * magic_c[:, None]) >> 18
    c = km - r * ncc[:, None]
    idx0 = base + (s0 + 1) * L + (s1 + 1)
    idx_e = base + e0 * L + e1
    idx_rect = base[:, None] + (s0[:, None] + 1 + r) * L + (s1[:, None] + 1 + c)
    idxr = jnp.concatenate(
        [idx_rect, idx0[:, None], idx_e[:, None],
         jnp.broadcast_to(idx0[:, None], (N, 14))], axis=1)
    idxr = idxr.astype(jnp.int32).reshape(N * PADW)

    order_n = jnp.argsort(-n)
    topk = order_n[:KTC].astype(jnp.int32)
    tcflag = jnp.zeros((N,), jnp.int32).at[topk].set(1)
    v0 = s1[topk] + 1
    v0a = jnp.minimum(v0 & ~7, L - TCC)
    sp = jnp.stack(
        [topk // P, s0[topk] + 1, v0a,
         jnp.minimum(nr[topk], TCW), jnp.minimum(ncc[topk], TCW),
         topk, v0 - v0a, jnp.zeros_like(topk)],
        axis=1).astype(jnp.int32)

    nch = (n + CH - 1) // CH
    nch = jnp.where(tcflag > 0, 1, nch)
    order = jnp.argsort(-nch)
    ids = order.reshape(ppw, NW)
    flip = (jnp.arange(ppw) % 2 == 1)[:, None]
    ids = jnp.where(flip, ids[:, ::-1], ids)
    perm = ids.T.astype(jnp.int32)
    meta = jnp.concatenate([perm, nch[perm]], axis=1).astype(jnp.int32)
    meta = meta.reshape(NW * 16)

    rtc = _tc_rect(Table, sp, N, D)
    enc = _sc_encode(tf, ef, idxr, meta, N, D)

    loss, logits, pred = _classify(enc, rtc, tcflag.reshape(N, 1),
                                   W_lin.reshape(5, D, 4),
                                   b_lin.reshape(1, 4), labels.reshape(N, 1))
    return loss[0, 0], logits.reshape(B, P, 4), pred.reshape(B, P)


# speedup vs baseline: 1.0209x; 1.0209x over previous
import functools

import jax
import jax.numpy as jnp
from jax import lax
from jax.experimental import pallas as pl
from jax.experimental.pallas import tpu as pltpu
from jax.experimental.pallas import tpu_sc as plsc

NC, NS, LANES = 2, 16, 16
NW = NC * NS
CH = 64
WWIN = 30
MAXC = (WWIN * WWIN + CH - 1) // CH
PAD = MAXC * CH
PADW = PAD + 16
TCW = 29
TCC = 40
KTC = 48


def _sc_encode(tf, ef, idxr, meta, n_pairs, d):
    ppw = n_pairs // NW
    vpr = d // LANES
    grp = 16
    ngrp = vpr // grp
    mesh = plsc.VectorSubcoreMesh(core_axis_name="c", subcore_axis_name="s",
                                  num_cores=NC, num_subcores=NS)

    @functools.partial(
        pl.kernel,
        out_type=jax.ShapeDtypeStruct((n_pairs, 5, d), jnp.float32),
        mesh=mesh,
        scratch_types=[
            pltpu.VMEM((PADW,), jnp.int32),
            pltpu.VMEM((PADW,), jnp.int32),
            pltpu.VMEM((2 * CH, d), jnp.float32),
            pltpu.VMEM((2, d), jnp.float32),
            pltpu.VMEM((2, d), jnp.float32),
            pltpu.VMEM((2, d), jnp.float32),
            pltpu.VMEM((2, d), jnp.float32),
            pltpu.VMEM((1, d), jnp.float32),
            pltpu.VMEM((1, d), jnp.float32),
            pltpu.VMEM((16,), jnp.int32),
            pltpu.SemaphoreType.DMA,
            pltpu.SemaphoreType.DMA,
            pltpu.SemaphoreType.DMA,
            pltpu.SemaphoreType.DMA,
            pltpu.SemaphoreType.DMA,
            pltpu.SemaphoreType.DMA,
            pltpu.SemaphoreType.DMA,
        ],
    )
    def k(tf_h, ef_h, idxr_h, meta_h, enc_h,
          idx_a, idx_b, rows, cor_t_a, cor_t_b, cor_e_a, cor_e_b,
          racc_a, racc_b, meta_v,
          sem_idx_a, sem_idx_b, sem_ra, sem_rb, sem_cor,
          sem_out_a, sem_out_b):
        wid = lax.axis_index("s") * NC + lax.axis_index("c")
        pltpu.sync_copy(meta_h.at[pl.ds(wid * 16, 16)], meta_v)
        mv = meta_v[...]
        idx_bufs = (idx_a, idx_b)
        row_halves = (rows.at[pl.ds(0, CH)], rows.at[pl.ds(CH, CH)])
        cor_t = (cor_t_a, cor_t_b)
        cor_e = (cor_e_a, cor_e_b)
        racc = (racc_a, racc_b)
        sem_idx = (sem_idx_a, sem_idx_b)
        sem_rows = (sem_ra, sem_rb)
        sem_out = (sem_out_a, sem_out_b)
        neg = jnp.full((LANES,), -jnp.inf, jnp.float32)

        def out_writes(jb, p):
            return (
                pltpu.make_async_copy(cor_t[jb], enc_h.at[p, pl.ds(0, 2)],
                                      sem_out[jb]),
                pltpu.make_async_copy(racc[jb], enc_h.at[p, pl.ds(2, 1)],
                                      sem_out[jb]),
                pltpu.make_async_copy(cor_e[jb], enc_h.at[p, pl.ds(3, 2)],
                                      sem_out[jb]),
            )

        pltpu.async_copy(idxr_h.at[pl.ds(mv[0] * PADW, PADW)],
                         idx_bufs[0], sem_idx[0])
        pltpu.make_async_copy(idxr_h.at[pl.ds(mv[0] * PADW, PADW)],
                              idx_bufs[0], sem_idx[0]).wait()
        pltpu.async_copy(tf_h.at[idx_bufs[0].at[pl.ds(0, CH)]],
                         row_halves[0], sem_rows[0])
        gbase = jnp.int32(0)
        prev_p = [None, None]
        for j in range(ppw):
            jb = j % 2
            p = mv[j]
            cnt = mv[8 + j]
            idx_v = idx_bufs[jb]
            if j >= 2:
                for w in out_writes(jb, prev_p[jb]):
                    w.wait()
            pltpu.async_copy(tf_h.at[idx_v.at[pl.ds(PAD, 2)]],
                             cor_t[jb], sem_cor)
            pltpu.async_copy(ef_h.at[idx_v.at[pl.ds(PAD, 2)]],
                             cor_e[jb], sem_cor)
            if j + 1 < ppw:
                pltpu.async_copy(idxr_h.at[pl.ds(mv[j + 1] * PADW, PADW)],
                                 idx_bufs[(j + 1) % 2], sem_idx[(j + 1) % 2])
            for kk in range(vpr):
                racc[jb][0, pl.ds(kk * LANES, LANES)] = neg

            def accum(base):
                for g in range(ngrp):
                    off = g * grp * LANES
                    accs = tuple(racc[jb][0, pl.ds(off + kk * LANES, LANES)]
                                 for kk in range(grp))

                    def row_body(rr, a):
                        return tuple(
                            jnp.maximum(a[kk],
                                        rows[base + rr,
                                             pl.ds(off + kk * LANES, LANES)])
                            for kk in range(grp))

                    accs = lax.fori_loop(0, CH, row_body, accs)
                    for kk in range(grp):
                        racc[jb][0, pl.ds(off + kk * LANES, LANES)] = accs[kk]

            def chunk_body(ci, carry):
                parv = (gbase + ci) % 2
                for par in range(2):
                    @pl.when(parv == par)
                    def _():
                        cur, nxt = row_halves[par], row_halves[1 - par]
                        pltpu.make_async_copy(
                            tf_h.at[idx_v.at[pl.ds(ci * CH, CH)]],
                            cur, sem_rows[par]).wait()

                        @pl.when(ci + 1 < cnt)
                        def _():
                            pltpu.async_copy(
                                tf_h.at[idx_v.at[pl.ds((ci + 1) * CH, CH)]],
                                nxt, sem_rows[1 - par])

                        if j + 1 < ppw:
                            nidx = idx_bufs[(j + 1) % 2]

                            @pl.when(ci + 1 == cnt)
                            def _():
                                pltpu.make_async_copy(
                                    idxr_h.at[pl.ds(mv[j + 1] * PADW, PADW)],
                                    nidx, sem_idx[(j + 1) % 2]).wait()
                                pltpu.async_copy(
                                    tf_h.at[nidx.at[pl.ds(0, CH)]],
                                    nxt, sem_rows[1 - par])

                accum(parv * CH)
                return carry

            lax.fori_loop(0, cnt, chunk_body, 0)
            gbase = (gbase + cnt) % 2

            pltpu.make_async_copy(tf_h.at[idx_v.at[pl.ds(PAD, 2)]],
                                  cor_t[jb], sem_cor).wait()
            pltpu.make_async_copy(ef_h.at[idx_v.at[pl.ds(PAD, 2)]],
                                  cor_e[jb], sem_cor).wait()
            for w in out_writes(jb, p):
                w.start()
            prev_p[jb] = p

        for jb in range(2):
            for w in out_writes(jb, prev_p[jb]):
                w.wait()

    return k(tf, ef, idxr, meta)


def _tc_rect(Table, sp, n_pairs, d):
    K = sp.shape[0]

    def body(sp_ref, tf_ref, out_ref, buf, sems):
        i = pl.program_id(0)

        def desc(slot, idx):
            b = sp_ref[idx, 0]
            u0 = sp_ref[idx, 1]
            v0a = pl.multiple_of(sp_ref[idx, 2], 8)
            return pltpu.make_async_copy(
                tf_ref.at[b, pl.ds(u0, TCW), pl.ds(v0a, TCC), :],
                buf.at[slot], sems.at[slot])

        @pl.when(i == 0)
        def _():
            desc(0, 0).start()

        @pl.when(i + 1 < K)
        def _():
            desc((i + 1) % 2, i + 1).start()

        slot = i % 2
        desc(slot, i).wait()
        nr = sp_ref[i, 3]
        ncv = sp_ref[i, 4]
        coff = sp_ref[i, 6]
        blk = buf[slot]
        rmask = lax.broadcasted_iota(jnp.int32, (TCW, TCC, 1), 0) < nr
        ci = lax.broadcasted_iota(jnp.int32, (TCW, TCC, 1), 1)
        cmask = (ci >= coff) & (ci < coff + ncv)
        masked = jnp.where(rmask & cmask, blk, -jnp.inf)
        out_ref[...] = jnp.max(masked, axis=(0, 1))[None, None, :]

    grid_spec = pltpu.PrefetchScalarGridSpec(
        num_scalar_prefetch=1,
        grid=(K,),
        in_specs=[pl.BlockSpec(memory_space=pl.ANY)],
        out_specs=pl.BlockSpec((1, 1, d), lambda i, sp_ref: (sp_ref[i, 5], 0, 0)),
        scratch_shapes=[pltpu.VMEM((2, TCW, TCC, d), jnp.float32),
                        pltpu.SemaphoreType.DMA((2,))],
    )
    return pl.pallas_call(
        body, grid_spec=grid_spec,
        out_shape=jax.ShapeDtypeStruct((n_pairs, 1, d), jnp.float32))(sp, Table)


def _classify(enc3, rtc, tcf, w3, b2, lab2):
    n, nslab, _ = enc3.shape

    def body(enc_ref, rtc_ref, tcf_ref, w_ref, b_ref, lab_ref,
             loss_ref, logits_ref, pred_ref):
        logits = b_ref[...]
        for rr in range(nslab):
            x = enc_ref[:, rr, :]
            if rr == 2:
                x = jnp.where(tcf_ref[...] > 0, rtc_ref[:, 0, :], x)
            logits = logits + jnp.dot(x, w_ref[rr],
                                      preferred_element_type=jnp.float32)
        m = jnp.max(logits, axis=1, keepdims=True)
        lse = jnp.log(jnp.sum(jnp.exp(logits - m), axis=1, keepdims=True)) + m
        logp = logits - lse
        lab = lab_ref[...]
        valid = lab >= 0
        safe = jnp.where(valid, lab, 0)
        cols = lax.broadcasted_iota(jnp.int32, logits.shape, 1)
        nll = -jnp.sum(jnp.where(cols == safe, logp, 0.0), axis=1, keepdims=True)
        cnt = jnp.maximum(jnp.sum(valid.astype(jnp.int32)), 1).astype(jnp.float32)
        loss_ref[...] = (jnp.sum(jnp.where(valid, nll, 0.0)) / cnt).reshape(1, 1)
        logits_ref[...] = logits
        pred_ref[...] = jnp.argmax(logits, axis=1).astype(jnp.int32)[:, None]

    return pl.pallas_call(
        body,
        out_shape=(
            jax.ShapeDtypeStruct((1, 1), jnp.float32),
            jax.ShapeDtypeStruct((n, 4), jnp.float32),
            jax.ShapeDtypeStruct((n, 1), jnp.int32),
        ),
    )(enc3, rtc, tcf, w3, b2, lab2)


def kernel(Table, table_edge, pairs, labels, W_lin, b_lin):
    B, L, _, D = Table.shape
    P = pairs.shape[1]
    N = B * P
    ppw = N // NW

    tf = Table.reshape(B * L * L, D)
    ef = table_edge.reshape(B * L * L, D)

    pf = pairs.reshape(N, 4).astype(jnp.int32)
    s0, e0, s1, e1 = pf[:, 0], pf[:, 1], pf[:, 2], pf[:, 3]
    base = (jnp.arange(N, dtype=jnp.int32) // P) * (L * L)
    nr = jnp.clip(e0 - s0, 1, WWIN)
    ncc = jnp.clip(e1 - s1, 1, WWIN)
    n = nr * ncc
    kk = jnp.arange(PAD, dtype=jnp.int32)[None, :]
    magic_n = ((1 << 21) + n - 1) // n
    km = kk - n[:, None] * ((kk * magic_n[:, None]) >> 21)
    magic_c = ((1 << 18) + ncc - 1) // ncc
    r = (km * magic_c[:, None]) >> 18
    c = km - r * ncc[:, None]
    idx0 = base + (s0 + 1) * L + (s1 + 1)
    idx_e = base + e0 * L + e1
    idx_rect = base[:, None] + (s0[:, None] + 1 + r) * L + (s1[:, None] + 1 + c)
    idxr = jnp.concatenate(
        [idx_rect, idx0[:, None], idx_e[:, None],
         jnp.broadcast_to(idx0[:, None], (N, 14))], axis=1)
    idxr = idxr.astype(jnp.int32).reshape(N * PADW)

    order_n = jnp.argsort(-n)
    topk = order_n[:KTC].astype(jnp.int32)
    tcflag = jnp.zeros((N,), jnp.int32).at[topk].set(1)
    v0 = s1[topk] + 1
    v0a = jnp.minimum(v0 & ~7, L - TCC)
    sp = jnp.stack(
        [topk // P, s0[topk] + 1, v0a,
         jnp.minimum(nr[topk], TCW), jnp.minimum(ncc[topk], TCW),
         topk, v0 - v0a, jnp.zeros_like(topk)],
        axis=1).astype(jnp.int32)

    nch = (n + CH - 1) // CH
    nch = jnp.where(tcflag > 0, 1, nch)
    order = jnp.concatenate([order_n[KTC:], order_n[:KTC]]).astype(jnp.int32)
    ids = order.reshape(ppw, NW)
    flip = (jnp.arange(ppw) % 2 == 1)[:, None]
    ids = jnp.where(flip, ids[:, ::-1], ids)
    perm = ids.T.astype(jnp.int32)
    meta = jnp.concatenate([perm, nch[perm]], axis=1).astype(jnp.int32)
    meta = meta.reshape(NW * 16)

    rtc = _tc_rect(Table, sp, N, D)
    enc = _sc_encode(tf, ef, idxr, meta, N, D)

    loss, logits, pred = _classify(enc, rtc, tcflag.reshape(N, 1),
                                   W_lin.reshape(5, D, 4),
                                   b_lin.reshape(1, 4), labels.reshape(N, 1))
    return loss[0, 0], logits.reshape(B, P, 4), pred.reshape(B, P)


# speedup vs baseline: 1.0496x; 1.0281x over previous
import functools

import jax
import jax.numpy as jnp
from jax import lax
from jax.experimental import pallas as pl
from jax.experimental.pallas import tpu as pltpu
from jax.experimental.pallas import tpu_sc as plsc

NC, NS, LANES = 2, 16, 16
NW = NC * NS
CH = 72
WWIN = 30
MAXC = (WWIN * WWIN + CH - 1) // CH
PAD = MAXC * CH
PADW = PAD + 16


def _sc_encode(tf, ef, idxr, meta, n_pairs, d):
    ppw = n_pairs // NW
    vpr = d // LANES
    grp = 16
    ngrp = vpr // grp
    mesh = plsc.VectorSubcoreMesh(core_axis_name="c", subcore_axis_name="s",
                                  num_cores=NC, num_subcores=NS)

    @functools.partial(
        pl.kernel,
        out_type=jax.ShapeDtypeStruct((n_pairs, 5, d), jnp.float32),
        mesh=mesh,
        scratch_types=[
            pltpu.VMEM((PADW,), jnp.int32),
            pltpu.VMEM((PADW,), jnp.int32),
            pltpu.VMEM((CH, d), jnp.float32),
            pltpu.VMEM((CH, d), jnp.float32),
            pltpu.VMEM((2, d), jnp.float32),
            pltpu.VMEM((2, d), jnp.float32),
            pltpu.VMEM((2, d), jnp.float32),
            pltpu.VMEM((2, d), jnp.float32),
            pltpu.VMEM((1, d), jnp.float32),
            pltpu.VMEM((1, d), jnp.float32),
            pltpu.VMEM((16,), jnp.int32),
            pltpu.SemaphoreType.DMA,
            pltpu.SemaphoreType.DMA,
            pltpu.SemaphoreType.DMA,
            pltpu.SemaphoreType.DMA,
            pltpu.SemaphoreType.DMA,
            pltpu.SemaphoreType.DMA,
            pltpu.SemaphoreType.DMA,
        ],
    )
    def k(tf_h, ef_h, idxr_h, meta_h, enc_h,
          idx_a, idx_b, rows_a, rows_b, cor_t_a, cor_t_b, cor_e_a, cor_e_b,
          racc_a, racc_b, meta_v,
          sem_idx_a, sem_idx_b, sem_ra, sem_rb, sem_cor,
          sem_out_a, sem_out_b):
        wid = lax.axis_index("s") * NC + lax.axis_index("c")
        pltpu.sync_copy(meta_h.at[pl.ds(wid * 16, 16)], meta_v)
        mv = meta_v[...]
        idx_bufs = (idx_a, idx_b)
        row_bufs = (rows_a, rows_b)
        cor_t = (cor_t_a, cor_t_b)
        cor_e = (cor_e_a, cor_e_b)
        racc = (racc_a, racc_b)
        sem_idx = (sem_idx_a, sem_idx_b)
        sem_rows = (sem_ra, sem_rb)
        sem_out = (sem_out_a, sem_out_b)
        neg = jnp.full((LANES,), -jnp.inf, jnp.float32)

        def out_writes(jb, p):
            return (
                pltpu.make_async_copy(cor_t[jb], enc_h.at[p, pl.ds(0, 2)],
                                      sem_out[jb]),
                pltpu.make_async_copy(racc[jb], enc_h.at[p, pl.ds(2, 1)],
                                      sem_out[jb]),
                pltpu.make_async_copy(cor_e[jb], enc_h.at[p, pl.ds(3, 2)],
                                      sem_out[jb]),
            )

        pltpu.async_copy(idxr_h.at[pl.ds(mv[0] * PADW, PADW)],
                         idx_bufs[0], sem_idx[0])
        pltpu.make_async_copy(idxr_h.at[pl.ds(mv[0] * PADW, PADW)],
                              idx_bufs[0], sem_idx[0]).wait()
        pltpu.async_copy(tf_h.at[idx_bufs[0].at[pl.ds(0, CH)]],
                         row_bufs[0], sem_rows[0])
        gbase = jnp.int32(0)
        prev_p = [None, None]
        for j in range(ppw):
            jb = j % 2
            p = mv[j]
            cnt = mv[8 + j]
            idx_v = idx_bufs[jb]
            if j >= 2:
                for w in out_writes(jb, prev_p[jb]):
                    w.wait()
            pltpu.async_copy(tf_h.at[idx_v.at[pl.ds(PAD, 2)]],
                             cor_t[jb], sem_cor)
            pltpu.async_copy(ef_h.at[idx_v.at[pl.ds(PAD, 2)]],
                             cor_e[jb], sem_cor)
            if j + 1 < ppw:
                pltpu.async_copy(idxr_h.at[pl.ds(mv[j + 1] * PADW, PADW)],
                                 idx_bufs[(j + 1) % 2], sem_idx[(j + 1) % 2])
            for kk in range(vpr):
                racc[jb][0, pl.ds(kk * LANES, LANES)] = neg

            def accum(rows):
                for g in range(ngrp):
                    off = g * grp * LANES
                    accs = tuple(racc[jb][0, pl.ds(off + kk * LANES, LANES)]
                                 for kk in range(grp))

                    def row_body(rr, a):
                        return tuple(
                            jnp.maximum(a[kk],
                                        rows[rr, pl.ds(off + kk * LANES, LANES)])
                            for kk in range(grp))

                    accs = lax.fori_loop(0, CH, row_body, accs)
                    for kk in range(grp):
                        racc[jb][0, pl.ds(off + kk * LANES, LANES)] = accs[kk]

            def chunk_body(ci, carry):
                for par in range(2):
                    @pl.when((gbase + ci) % 2 == par)
                    def _():
                        cur, nxt = row_bufs[par], row_bufs[1 - par]
                        pltpu.make_async_copy(
                            tf_h.at[idx_v.at[pl.ds(ci * CH, CH)]],
                            cur, sem_rows[par]).wait()

                        @pl.when(ci + 1 < cnt)
                        def _():
                            pltpu.async_copy(
                                tf_h.at[idx_v.at[pl.ds((ci + 1) * CH, CH)]],
                                nxt, sem_rows[1 - par])

                        if j + 1 < ppw:
                            nidx = idx_bufs[(j + 1) % 2]

                            @pl.when(ci + 1 == cnt)
                            def _():
                                pltpu.make_async_copy(
                                    idxr_h.at[pl.ds(mv[j + 1] * PADW, PADW)],
                                    nidx, sem_idx[(j + 1) % 2]).wait()
                                pltpu.async_copy(
                                    tf_h.at[nidx.at[pl.ds(0, CH)]],
                                    nxt, sem_rows[1 - par])

                        accum(cur)
                return carry

            lax.fori_loop(0, cnt, chunk_body, 0)
            gbase = (gbase + cnt) % 2

            pltpu.make_async_copy(tf_h.at[idx_v.at[pl.ds(PAD, 2)]],
                                  cor_t[jb], sem_cor).wait()
            pltpu.make_async_copy(ef_h.at[idx_v.at[pl.ds(PAD, 2)]],
                                  cor_e[jb], sem_cor).wait()
            for w in out_writes(jb, p):
                w.start()
            prev_p[jb] = p

        for jb in range(2):
            for w in out_writes(jb, prev_p[jb]):
                w.wait()

    return k(tf, ef, idxr, meta)


def _classify(enc3, w3, b2, lab2):
    n, nslab, _ = enc3.shape

    def body(enc_ref, w_ref, b_ref, lab_ref, loss_ref, logits_ref, pred_ref):
        logits = b_ref[...]
        for rr in range(nslab):
            logits = logits + jnp.dot(enc_ref[:, rr, :], w_ref[rr],
                                      preferred_element_type=jnp.float32)
        m = jnp.max(logits, axis=1, keepdims=True)
        lse = jnp.log(jnp.sum(jnp.exp(logits - m), axis=1, keepdims=True)) + m
        logp = logits - lse
        lab = lab_ref[...]
        valid = lab >= 0
        safe = jnp.where(valid, lab, 0)
        cols = lax.broadcasted_iota(jnp.int32, logits.shape, 1)
        nll = -jnp.sum(jnp.where(cols == safe, logp, 0.0), axis=1, keepdims=True)
        cnt = jnp.maximum(jnp.sum(valid.astype(jnp.int32)), 1).astype(jnp.float32)
        loss_ref[...] = (jnp.sum(jnp.where(valid, nll, 0.0)) / cnt).reshape(1, 1)
        logits_ref[...] = logits
        pred_ref[...] = jnp.argmax(logits, axis=1).astype(jnp.int32)[:, None]

    return pl.pallas_call(
        body,
        out_shape=(
            jax.ShapeDtypeStruct((1, 1), jnp.float32),
            jax.ShapeDtypeStruct((n, 4), jnp.float32),
            jax.ShapeDtypeStruct((n, 1), jnp.int32),
        ),
    )(enc3, w3, b2, lab2)


def kernel(Table, table_edge, pairs, labels, W_lin, b_lin):
    B, L, _, D = Table.shape
    P = pairs.shape[1]
    N = B * P
    ppw = N // NW

    tf = Table.reshape(B * L * L, D)
    ef = table_edge.reshape(B * L * L, D)

    pf = pairs.reshape(N, 4).astype(jnp.int32)
    s0, e0, s1, e1 = pf[:, 0], pf[:, 1], pf[:, 2], pf[:, 3]
    base = (jnp.arange(N, dtype=jnp.int32) // P) * (L * L)
    nr = jnp.clip(e0 - s0, 1, WWIN)
    ncc = jnp.clip(e1 - s1, 1, WWIN)
    n = nr * ncc
    kk = jnp.arange(PAD, dtype=jnp.int32)[None, :]
    magic_n = ((1 << 21) + n - 1) // n
    km = kk - n[:, None] * ((kk * magic_n[:, None]) >> 21)
    magic_c = ((1 << 18) + ncc - 1) // ncc
    r = (km * magic_c[:, None]) >> 18
    c = km - r * ncc[:, None]
    idx0 = base + (s0 + 1) * L + (s1 + 1)
    idx_e = base + e0 * L + e1
    idx_rect = base[:, None] + (s0[:, None] + 1 + r) * L + (s1[:, None] + 1 + c)
    idxr = jnp.concatenate(
        [idx_rect, idx0[:, None], idx_e[:, None],
         jnp.broadcast_to(idx0[:, None], (N, 14))], axis=1)
    idxr = idxr.astype(jnp.int32).reshape(N * PADW)

    nch = (n + CH - 1) // CH
    order = jnp.argsort(-nch)
    ids = order.reshape(ppw, NW)
    flip = (jnp.arange(ppw) % 2 == 1)[:, None]
    ids = jnp.where(flip, ids[:, ::-1], ids)
    perm = ids.T.astype(jnp.int32)
    meta = jnp.concatenate([perm, nch[perm]], axis=1).astype(jnp.int32)
    meta = meta.reshape(NW * 16)

    enc = _sc_encode(tf, ef, idxr, meta, N, D)

    loss, logits, pred = _classify(enc, W_lin.reshape(5, D, 4),
                                   b_lin.reshape(1, 4), labels.reshape(N, 1))
    return loss[0, 0], logits.reshape(B, P, 4), pred.reshape(B, P)
